# baseline (speedup 1.0000x reference)
import functools

import jax
import jax.numpy as jnp
from jax import lax
from jax.experimental import pallas as pl
from jax.experimental.pallas import tpu as pltpu
from jax.experimental.pallas import tpu_sc as plsc

_N = 524288
_NC = 2
_NS = 16
_NW = _NC * _NS
_L = 16
_PER_W = _N // _NW
_UNROLL = 8


def _build_sc_clamp():
    mesh = plsc.VectorSubcoreMesh(core_axis_name="c", subcore_axis_name="s")

    @functools.partial(
        pl.kernel,
        mesh=mesh,
        out_type=jax.ShapeDtypeStruct((_N,), jnp.float32),
        scratch_types=[
            pltpu.VMEM((_PER_W,), jnp.float32),
            pltpu.VMEM((2 * _L,), jnp.float32),
        ],
    )
    def sc_clamp(x_hbm, bounds_hbm, out_hbm, buf, bounds_v):
        wid = lax.axis_index("s") * _NC + lax.axis_index("c")
        base = wid * _PER_W
        pltpu.sync_copy(bounds_hbm, bounds_v)
        pltpu.sync_copy(x_hbm.at[pl.ds(base, _PER_W)], buf)
        lo = bounds_v[pl.ds(0, _L)]
        hi = bounds_v[pl.ds(_L, _L)]

        def body(i, carry):
            off = i * (_L * _UNROLL)
            for j in range(_UNROLL):
                s = pl.ds(off + j * _L, _L)
                buf[s] = jnp.maximum(jnp.minimum(buf[s], hi), lo)
            return carry

        lax.fori_loop(0, _PER_W // (_L * _UNROLL), body, 0)
        pltpu.sync_copy(buf, out_hbm.at[pl.ds(base, _PER_W)])

    return sc_clamp


_sc_clamp = _build_sc_clamp()


def kernel(tensor_input, values):
    bounds = jnp.concatenate([
        jnp.broadcast_to(values[0], (_L,)),
        jnp.broadcast_to(values[-1], (_L,)),
    ])
    out = _sc_clamp(tensor_input.reshape(_N), bounds)
    return out.reshape(tensor_input.shape)


# speedup vs baseline: 1.0361x; 1.0361x over previous
import functools

import jax
import jax.numpy as jnp
from jax import lax
from jax.experimental import pallas as pl
from jax.experimental.pallas import tpu as pltpu
from jax.experimental.pallas import tpu_sc as plsc

_N = 524288
_NC = 2
_NS = 16
_NW = _NC * _NS
_L = 16
_PER_W = _N // _NW
_UNROLL = 8


def _build_sc_clamp():
    mesh = plsc.VectorSubcoreMesh(core_axis_name="c", subcore_axis_name="s")

    @functools.partial(
        pl.kernel,
        mesh=mesh,
        out_type=jax.ShapeDtypeStruct((_N,), jnp.float32),
        scratch_types=[
            pltpu.VMEM((_PER_W,), jnp.float32),
            pltpu.VMEM((64,), jnp.float32),
        ],
    )
    def sc_clamp(x_hbm, vals_hbm, out_hbm, buf, vals_v):
        wid = lax.axis_index("s") * _NC + lax.axis_index("c")
        base = wid * _PER_W
        pltpu.sync_copy(vals_hbm, vals_v)
        pltpu.sync_copy(x_hbm.at[pl.ds(base, _PER_W)], buf)
        lo = jnp.full((_L,), vals_v[pl.ds(0, _L)][0], jnp.float32)
        hi = jnp.full((_L,), vals_v[pl.ds(48, _L)][_L - 1], jnp.float32)

        def body(i, carry):
            off = i * (_L * _UNROLL)
            for j in range(_UNROLL):
                s = pl.ds(off + j * _L, _L)
                buf[s] = jnp.maximum(jnp.minimum(buf[s], hi), lo)
            return carry

        lax.fori_loop(0, _PER_W // (_L * _UNROLL), body, 0)
        pltpu.sync_copy(buf, out_hbm.at[pl.ds(base, _PER_W)])

    return sc_clamp


_sc_clamp = _build_sc_clamp()


def kernel(tensor_input, values):
    out = _sc_clamp(tensor_input.reshape(_N), values)
    return out.reshape(tensor_input.shape)


# speedup vs baseline: 1.0720x; 1.0347x over previous
import functools

import jax
import jax.numpy as jnp
from jax import lax
from jax.experimental import pallas as pl
from jax.experimental.pallas import tpu as pltpu
from jax.experimental.pallas import tpu_sc as plsc

_N = 524288
_NC = 2
_NS = 16
_NW = _NC * _NS
_L = 16
_PER_W = _N // _NW
_NCHUNK = 4
_CHUNK = _PER_W // _NCHUNK
_UNROLL = 8


def _build_sc_clamp():
    mesh = plsc.VectorSubcoreMesh(core_axis_name="c", subcore_axis_name="s")

    @functools.partial(
        pl.kernel,
        mesh=mesh,
        out_type=jax.ShapeDtypeStruct((_N,), jnp.float32),
        scratch_types=[
            pltpu.VMEM((_PER_W,), jnp.float32),
            pltpu.VMEM((64,), jnp.float32),
            pltpu.SemaphoreType.DMA,
            pltpu.SemaphoreType.DMA,
            pltpu.SemaphoreType.DMA,
            pltpu.SemaphoreType.DMA,
            pltpu.SemaphoreType.DMA,
            pltpu.SemaphoreType.DMA,
        ],
    )
    def sc_clamp(x_hbm, vals_hbm, out_hbm, buf, vals_v,
                 s0, s1, s2, s3, vsem, osem):
        wid = lax.axis_index("s") * _NC + lax.axis_index("c")
        base = wid * _PER_W
        in_sems = (s0, s1, s2, s3)
        vcopy = pltpu.async_copy(vals_hbm, vals_v, vsem)
        in_copies = []
        for c in range(_NCHUNK):
            off = c * _CHUNK
            in_copies.append(pltpu.async_copy(
                x_hbm.at[pl.ds(base + off, _CHUNK)],
                buf.at[pl.ds(off, _CHUNK)], in_sems[c]))
        vcopy.wait()
        lo = jnp.full((_L,), vals_v[pl.ds(0, _L)][0], jnp.float32)
        hi = jnp.full((_L,), vals_v[pl.ds(48, _L)][_L - 1], jnp.float32)

        out_copies = []
        for c in range(_NCHUNK):
            off = c * _CHUNK
            in_copies[c].wait()

            def body(i, carry, off=off):
                o = off + i * (_L * _UNROLL)
                for j in range(_UNROLL):
                    s = pl.ds(o + j * _L, _L)
                    buf[s] = jnp.maximum(jnp.minimum(buf[s], hi), lo)
                return carry

            lax.fori_loop(0, _CHUNK // (_L * _UNROLL), body, 0)
            out_copies.append(pltpu.async_copy(
                buf.at[pl.ds(off, _CHUNK)],
                out_hbm.at[pl.ds(base + off, _CHUNK)], osem))
        for cp in out_copies:
            cp.wait()

    return sc_clamp


_sc_clamp = _build_sc_clamp()


def kernel(tensor_input, values):
    out = _sc_clamp(tensor_input.reshape(_N), values)
    return out.reshape(tensor_input.shape)


# speedup vs baseline: 1.0753x; 1.0031x over previous
import functools

import jax
import jax.numpy as jnp
from jax import lax
from jax.experimental import pallas as pl
from jax.experimental.pallas import tpu as pltpu
from jax.experimental.pallas import tpu_sc as plsc

_N = 524288
_NC = 2
_NS = 16
_L = 16
_UNROLL = 8

_A_PER_TILE = 14336
_A_CHUNK = _A_PER_TILE // 2
_A_TOTAL = _A_PER_TILE * _NC * _NS
_B_CHUNK = (_N - _A_TOTAL) // _NS


def _clamp_loop(buf, off, count, lo, hi):
    def body(i, carry):
        o = off + i * (_L * _UNROLL)
        for j in range(_UNROLL):
            s = pl.ds(o + j * _L, _L)
            buf[s] = jnp.maximum(jnp.minimum(buf[s], hi), lo)
        return carry

    lax.fori_loop(0, count // (_L * _UNROLL), body, 0)


def _build_sc_clamp():
    mesh = plsc.VectorSubcoreMesh(core_axis_name="c", subcore_axis_name="s")

    @functools.partial(
        pl.kernel,
        mesh=mesh,
        out_type=jax.ShapeDtypeStruct((_N,), jnp.float32),
        scratch_types=[
            pltpu.VMEM((_A_PER_TILE + _B_CHUNK,), jnp.float32),
            pltpu.VMEM((64,), jnp.float32),
            pltpu.SemaphoreType.DMA,
            pltpu.SemaphoreType.DMA,
            pltpu.SemaphoreType.DMA,
            pltpu.SemaphoreType.DMA,
            pltpu.SemaphoreType.DMA,
        ],
    )
    def sc_clamp(x_hbm, vals_hbm, out_hbm, buf, vals_v,
                 sa0, sa1, sb, vsem, osem):
        cid = lax.axis_index("c")
        wid = lax.axis_index("s") * _NC + cid
        a_base = wid * _A_PER_TILE
        b_base = _A_TOTAL + lax.axis_index("s") * _B_CHUNK
        b_off = 2 * _A_CHUNK

        vcopy = pltpu.async_copy(vals_hbm, vals_v, vsem)
        in_a = []
        for k in range(2):
            in_a.append(pltpu.async_copy(
                x_hbm.at[pl.ds(a_base + k * _A_CHUNK, _A_CHUNK)],
                buf.at[pl.ds(k * _A_CHUNK, _A_CHUNK)], (sa0, sa1)[k]))

        @pl.when(cid == 1)
        def _():
            pltpu.async_copy(
                x_hbm.at[pl.ds(b_base, _B_CHUNK)],
                buf.at[pl.ds(b_off, _B_CHUNK)], sb)

        vcopy.wait()
        lo = jnp.full((_L,), vals_v[pl.ds(0, _L)][0], jnp.float32)
        hi = jnp.full((_L,), vals_v[pl.ds(48, _L)][_L - 1], jnp.float32)

        out_a = []
        for k in range(2):
            in_a[k].wait()
            _clamp_loop(buf, k * _A_CHUNK, _A_CHUNK, lo, hi)
            out_a.append(pltpu.async_copy(
                buf.at[pl.ds(k * _A_CHUNK, _A_CHUNK)],
                out_hbm.at[pl.ds(a_base + k * _A_CHUNK, _A_CHUNK)], osem))

        @pl.when(cid == 1)
        def _():
            pltpu.make_async_copy(
                x_hbm.at[pl.ds(b_base, _B_CHUNK)],
                buf.at[pl.ds(b_off, _B_CHUNK)], sb).wait()
            _clamp_loop(buf, b_off, _B_CHUNK, lo, hi)
            pltpu.sync_copy(buf.at[pl.ds(b_off, _B_CHUNK)],
                            out_hbm.at[pl.ds(b_base, _B_CHUNK)])

        for cp in out_a:
            cp.wait()

    return sc_clamp


_sc_clamp = _build_sc_clamp()


def kernel(tensor_input, values):
    out = _sc_clamp(tensor_input.reshape(_N), values)
    return out.reshape(tensor_input.shape)
